# baseline (speedup 1.0000x reference)
import functools

import jax
import jax.numpy as jnp
from jax import lax
from jax.experimental import pallas as pl
from jax.experimental.pallas import tpu as pltpu
from jax.experimental.pallas import tpu_sc as plsc

N_NODES = 10000
D_IN = 128
D_HID = 16
D_OUT = 10

NC = 2
NS = 16
NW = NC * NS
BCH = 1024
KCH = 10
GB = 2
NGRP = KCH // GB
EPT = KCH * BCH
E_PAD = NW * EPT
NPAD = 10112
RPT = NPAD // NS

def _zero_my_slice(zbuf, table, s):
    def zrow(i, _):
        zbuf[i] = jnp.zeros((D_HID,), jnp.float32)
        return 0

    lax.fori_loop(0, RPT, zrow, 0)
    pltpu.sync_copy(zbuf, table.at[pl.ds(s * RPT, RPT)])


@functools.cache
def _make_sc_degree():
    mesh = plsc.VectorSubcoreMesh(core_axis_name="c", subcore_axis_name="s",
                                  num_cores=NC, num_subcores=NS)

    @functools.partial(
        pl.kernel,
        mesh=mesh,
        out_type=jax.ShapeDtypeStruct((NC, NPAD, D_HID), jnp.float32),
        scratch_types=[
            pltpu.VMEM((KCH, BCH), jnp.int32),
            pltpu.VMEM((BCH, D_HID), jnp.float32),
            pltpu.VMEM((RPT, D_HID), jnp.float32),
            pltpu.VMEM_SHARED((NPAD, D_HID), jnp.float32),
        ],
        compiler_params=pltpu.CompilerParams(use_tc_tiling_on_sc=False),
    )
    def sc_degree(dstp, out, didx, rows, zbuf, table):
        c = lax.axis_index("c")
        s = lax.axis_index("s")
        wid = s * NC + c
        _zero_my_slice(zbuf, table, s)

        def orow(i, _):
            rows[i] = jnp.ones((D_HID,), jnp.float32)
            return 0

        lax.fori_loop(0, BCH, orow, 0)
        pltpu.sync_copy(dstp.at[wid], didx)
        plsc.subcore_barrier()

        def chunk(j, _):
            pltpu.sync_copy(rows, table.at[didx.at[j]], add=True)
            return 0

        lax.fori_loop(0, KCH, chunk, 0)
        plsc.subcore_barrier()
        pltpu.sync_copy(table.at[pl.ds(s * RPT, RPT)],
                        out.at[c, pl.ds(s * RPT, RPT)])

    return sc_degree


@functools.cache
def _make_sc_scatter():
    mesh = plsc.VectorSubcoreMesh(core_axis_name="c", subcore_axis_name="s",
                                  num_cores=NC, num_subcores=NS)

    @functools.partial(
        pl.kernel,
        mesh=mesh,
        out_type=jax.ShapeDtypeStruct((NC, NPAD, D_HID), jnp.float32),
        scratch_types=[
            pltpu.VMEM((KCH, BCH), jnp.int32),
            pltpu.VMEM((KCH, BCH), jnp.int32),
            pltpu.VMEM((2, GB, BCH, D_HID), jnp.float32),
            pltpu.VMEM((RPT, D_HID), jnp.float32),
            pltpu.VMEM_SHARED((NPAD, D_HID), jnp.float32),
            pltpu.SemaphoreType.DMA,
        ],
        compiler_params=pltpu.CompilerParams(use_tc_tiling_on_sc=False),
    )
    def sc_scatter(g_hbm, srcp, dstp, out, sidx, didx, rows, zbuf, table,
                   gsem):
        c = lax.axis_index("c")
        s = lax.axis_index("s")
        wid = s * NC + c
        _zero_my_slice(zbuf, table, s)
        pltpu.sync_copy(srcp.at[wid], sidx)
        pltpu.sync_copy(dstp.at[wid], didx)
        plsc.subcore_barrier()

        for b in range(GB):
            pltpu.async_copy(g_hbm.at[sidx.at[b]], rows.at[0, b], gsem)

        def group(g, _):
            h = lax.rem(g, 2)
            for b in range(GB):
                pltpu.make_async_copy(
                    g_hbm.at[pl.ds(0, BCH)], rows.at[h, b], gsem).wait()

            @pl.when(g + 1 < NGRP)
            def _():
                for b in range(GB):
                    pltpu.async_copy(
                        g_hbm.at[sidx.at[(g + 1) * GB + b]],
                        rows.at[1 - h, b], gsem)

            for b in range(GB):
                pltpu.sync_copy(rows.at[h, b],
                                table.at[didx.at[g * GB + b]], add=True)
            return 0

        lax.fori_loop(0, NGRP, group, 0)
        plsc.subcore_barrier()
        pltpu.sync_copy(table.at[pl.ds(s * RPT, RPT)],
                        out.at[c, pl.ds(s * RPT, RPT)])

    return sc_scatter


_BLK = 2000
_GRID = N_NODES // _BLK


def _mm1_body(x_ref, w_ref, o_ref):
    o_ref[...] = jnp.dot(x_ref[...], w_ref[...],
                         preferred_element_type=jnp.float32)


def _tc_matmul1(x, W1):
    return pl.pallas_call(
        _mm1_body,
        grid=(_GRID,),
        in_specs=[
            pl.BlockSpec((_BLK, D_IN), lambda i: (i, 0)),
            pl.BlockSpec((D_IN, D_HID), lambda i: (0, 0)),
        ],
        out_specs=pl.BlockSpec((_BLK, D_HID), lambda i: (i, 0)),
        out_shape=jax.ShapeDtypeStruct((N_NODES, D_HID), jnp.float32),
    )(x, W1)


def _norm_body(da_ref, db_ref, h_ref, dinv_ref, g_ref):
    dinv = lax.rsqrt(da_ref[...] + db_ref[...] + 1.0)
    dinv_ref[...] = dinv
    g_ref[...] = h_ref[...] * dinv


def _tc_norm(degA, degB, h1):
    blk = pl.BlockSpec((_BLK, D_HID), lambda i: (i, 0))
    return pl.pallas_call(
        _norm_body,
        grid=(_GRID,),
        in_specs=[blk, blk, blk],
        out_specs=[blk, blk],
        out_shape=[
            jax.ShapeDtypeStruct((N_NODES, D_HID), jnp.float32),
            jax.ShapeDtypeStruct((N_NODES, D_HID), jnp.float32),
        ],
    )(degA, degB, h1)


def _layer2_body(ra_ref, rb_ref, g1_ref, dinv_ref, b1_ref, w2_ref, g2_ref):
    h2 = jnp.maximum(
        dinv_ref[...] * (ra_ref[...] + rb_ref[...] + g1_ref[...]) + b1_ref[...],
        0.0)
    g2_ref[...] = jnp.dot(h2, w2_ref[...],
                          preferred_element_type=jnp.float32) * dinv_ref[...]


def _tc_layer2(rawA, rawB, g1, dinv, b1r, W2p):
    blk = pl.BlockSpec((_BLK, D_HID), lambda i: (i, 0))
    vec = pl.BlockSpec((1, D_HID), lambda i: (0, 0))
    mat = pl.BlockSpec((D_HID, D_HID), lambda i: (0, 0))
    return pl.pallas_call(
        _layer2_body,
        grid=(_GRID,),
        in_specs=[blk, blk, blk, blk, vec, mat],
        out_specs=blk,
        out_shape=jax.ShapeDtypeStruct((N_NODES, D_HID), jnp.float32),
    )(rawA, rawB, g1, dinv, b1r, W2p)


def _final_body(ra_ref, rb_ref, g2_ref, dinv_ref, b2_ref, o_ref):
    o_ref[...] = (dinv_ref[...] * (ra_ref[...] + rb_ref[...] + g2_ref[...])
                  + b2_ref[...])


def _tc_final(rawA, rawB, g2, dinv, b2p):
    blk = pl.BlockSpec((_BLK, D_HID), lambda i: (i, 0))
    vec = pl.BlockSpec((1, D_HID), lambda i: (0, 0))
    return pl.pallas_call(
        _final_body,
        grid=(_GRID,),
        in_specs=[blk, blk, blk, blk, vec],
        out_specs=blk,
        out_shape=jax.ShapeDtypeStruct((N_NODES, D_HID), jnp.float32),
    )(rawA, rawB, g2, dinv, b2p)


def kernel(x, edge_index, W1, b1, W2, b2):
    e = edge_index.shape[1]
    pad = E_PAD - e
    src_p = jnp.concatenate(
        [edge_index[0], jnp.zeros((pad,), edge_index.dtype)]).reshape(
            NW, KCH, BCH)
    dst_p = jnp.concatenate(
        [edge_index[1], jnp.full((pad,), N_NODES, edge_index.dtype)]).reshape(
            NW, KCH, BCH)
    W2p = jnp.zeros((D_HID, D_HID), jnp.float32).at[:, :D_OUT].set(W2)
    b1r = b1.reshape(1, D_HID)
    b2p = jnp.zeros((1, D_HID), jnp.float32).at[0, :D_OUT].set(b2)

    sc_degree = _make_sc_degree()
    sc_scatter = _make_sc_scatter()

    deg = sc_degree(dst_p)
    h1 = _tc_matmul1(x, W1)
    dinv, g1 = _tc_norm(deg[0, :N_NODES], deg[1, :N_NODES], h1)
    raw1 = sc_scatter(g1, src_p, dst_p)
    g2 = _tc_layer2(raw1[0, :N_NODES], raw1[1, :N_NODES], g1, dinv, b1r, W2p)
    raw2 = sc_scatter(g2, src_p, dst_p)
    out16 = _tc_final(raw2[0, :N_NODES], raw2[1, :N_NODES], g2, dinv, b2p)
    return out16[:, :D_OUT]


# speedup vs baseline: 1.5557x; 1.5557x over previous
import functools

import jax
import jax.numpy as jnp
from jax import lax
from jax.experimental import pallas as pl
from jax.experimental.pallas import tpu as pltpu
from jax.experimental.pallas import tpu_sc as plsc

N_NODES = 10000
D_IN = 128
D_HID = 16
D_OUT = 10

NC = 2
NS = 16
NW = NC * NS
BCH = 1024
KCH = 10
GB = 2
NGRP = KCH // GB
EPT = KCH * BCH
E_PAD = NW * EPT
NPAD = 10112
RPT = NPAD // NS
GLAST = N_NODES - (NS - 1) * RPT

def _zero_my_slice(zbuf, table, s):
    def zrow(i, _):
        zbuf[i] = jnp.zeros((D_HID,), jnp.float32)
        return 0

    lax.fori_loop(0, RPT, zrow, 0)
    pltpu.sync_copy(zbuf, table.at[pl.ds(s * RPT, RPT)])


@functools.cache
def _make_sc_degree():
    mesh = plsc.VectorSubcoreMesh(core_axis_name="c", subcore_axis_name="s",
                                  num_cores=NC, num_subcores=NS)

    @functools.partial(
        pl.kernel,
        mesh=mesh,
        out_type=jax.ShapeDtypeStruct((NC, NPAD, D_HID), jnp.float32),
        scratch_types=[
            pltpu.VMEM((KCH, BCH), jnp.int32),
            pltpu.VMEM((BCH, D_HID), jnp.float32),
            pltpu.VMEM((RPT, D_HID), jnp.float32),
            pltpu.VMEM_SHARED((NPAD, D_HID), jnp.float32),
        ],
        compiler_params=pltpu.CompilerParams(use_tc_tiling_on_sc=False),
    )
    def sc_degree(dstp, out, didx, rows, zbuf, table):
        c = lax.axis_index("c")
        s = lax.axis_index("s")
        wid = s * NC + c
        _zero_my_slice(zbuf, table, s)

        def orow(i, _):
            rows[i] = jnp.ones((D_HID,), jnp.float32)
            return 0

        lax.fori_loop(0, BCH, orow, 0)
        pltpu.sync_copy(dstp.at[wid], didx)
        plsc.subcore_barrier()

        def chunk(j, _):
            pltpu.sync_copy(rows, table.at[didx.at[j]], add=True)
            return 0

        lax.fori_loop(0, KCH, chunk, 0)
        plsc.subcore_barrier()
        pltpu.sync_copy(table.at[pl.ds(s * RPT, RPT)],
                        out.at[c, pl.ds(s * RPT, RPT)])

    return sc_degree


@functools.cache
def _make_sc_scatter():
    mesh = plsc.VectorSubcoreMesh(core_axis_name="c", subcore_axis_name="s",
                                  num_cores=NC, num_subcores=NS)

    @functools.partial(
        pl.kernel,
        mesh=mesh,
        out_type=jax.ShapeDtypeStruct((NC, NPAD, D_HID), jnp.float32),
        scratch_types=[
            pltpu.VMEM((KCH, BCH), jnp.int32),
            pltpu.VMEM((KCH, BCH), jnp.int32),
            pltpu.VMEM((2, GB, BCH, D_HID), jnp.float32),
            pltpu.VMEM((RPT, D_HID), jnp.float32),
            pltpu.VMEM_SHARED((NPAD, D_HID), jnp.float32),
            pltpu.VMEM_SHARED((NPAD, D_HID), jnp.float32),
            pltpu.SemaphoreType.DMA,
        ],
        compiler_params=pltpu.CompilerParams(use_tc_tiling_on_sc=False),
    )
    def sc_scatter(g_hbm, srcp, dstp, out, sidx, didx, rows, zbuf, gshared,
                   table, gsem):
        c = lax.axis_index("c")
        s = lax.axis_index("s")
        wid = s * NC + c
        _zero_my_slice(zbuf, table, s)

        @pl.when(s < NS - 1)
        def _():
            pltpu.sync_copy(g_hbm.at[pl.ds(s * RPT, RPT)],
                            gshared.at[pl.ds(s * RPT, RPT)])

        @pl.when(s == NS - 1)
        def _():
            pltpu.sync_copy(g_hbm.at[pl.ds((NS - 1) * RPT, GLAST)],
                            gshared.at[pl.ds((NS - 1) * RPT, GLAST)])

        pltpu.sync_copy(srcp.at[wid], sidx)
        pltpu.sync_copy(dstp.at[wid], didx)
        plsc.subcore_barrier()

        for b in range(GB):
            pltpu.async_copy(gshared.at[sidx.at[b]], rows.at[0, b], gsem)

        def group(g, _):
            h = lax.rem(g, 2)
            for b in range(GB):
                pltpu.make_async_copy(
                    g_hbm.at[pl.ds(0, BCH)], rows.at[h, b], gsem).wait()

            @pl.when(g + 1 < NGRP)
            def _():
                for b in range(GB):
                    pltpu.async_copy(
                        gshared.at[sidx.at[(g + 1) * GB + b]],
                        rows.at[1 - h, b], gsem)

            for b in range(GB):
                pltpu.sync_copy(rows.at[h, b],
                                table.at[didx.at[g * GB + b]], add=True)
            return 0

        lax.fori_loop(0, NGRP, group, 0)
        plsc.subcore_barrier()
        pltpu.sync_copy(table.at[pl.ds(s * RPT, RPT)],
                        out.at[c, pl.ds(s * RPT, RPT)])

    return sc_scatter


_BLK = 2000
_GRID = N_NODES // _BLK


def _mm1_body(x_ref, w_ref, o_ref):
    o_ref[...] = jnp.dot(x_ref[...], w_ref[...],
                         preferred_element_type=jnp.float32)


def _tc_matmul1(x, W1):
    return pl.pallas_call(
        _mm1_body,
        grid=(_GRID,),
        in_specs=[
            pl.BlockSpec((_BLK, D_IN), lambda i: (i, 0)),
            pl.BlockSpec((D_IN, D_HID), lambda i: (0, 0)),
        ],
        out_specs=pl.BlockSpec((_BLK, D_HID), lambda i: (i, 0)),
        out_shape=jax.ShapeDtypeStruct((N_NODES, D_HID), jnp.float32),
    )(x, W1)


def _norm_body(da_ref, db_ref, h_ref, dinv_ref, g_ref):
    dinv = lax.rsqrt(da_ref[0] + db_ref[0] + 1.0)
    dinv_ref[...] = dinv
    g_ref[...] = h_ref[...] * dinv


def _tc_norm(deg, h1):
    blk = pl.BlockSpec((_BLK, D_HID), lambda i: (i, 0))
    pA = pl.BlockSpec((1, _BLK, D_HID), lambda i: (0, i, 0))
    pB = pl.BlockSpec((1, _BLK, D_HID), lambda i: (1, i, 0))
    return pl.pallas_call(
        _norm_body,
        grid=(_GRID,),
        in_specs=[pA, pB, blk],
        out_specs=[blk, blk],
        out_shape=[
            jax.ShapeDtypeStruct((N_NODES, D_HID), jnp.float32),
            jax.ShapeDtypeStruct((N_NODES, D_HID), jnp.float32),
        ],
    )(deg, deg, h1)


def _layer2_body(ra_ref, rb_ref, g1_ref, dinv_ref, b1_ref, w2_ref, g2_ref):
    h2 = jnp.maximum(
        dinv_ref[...] * (ra_ref[0] + rb_ref[0] + g1_ref[...]) + b1_ref[...],
        0.0)
    g2_ref[...] = jnp.dot(h2, w2_ref[...],
                          preferred_element_type=jnp.float32) * dinv_ref[...]


def _tc_layer2(raw, g1, dinv, b1r, W2p):
    blk = pl.BlockSpec((_BLK, D_HID), lambda i: (i, 0))
    pA = pl.BlockSpec((1, _BLK, D_HID), lambda i: (0, i, 0))
    pB = pl.BlockSpec((1, _BLK, D_HID), lambda i: (1, i, 0))
    vec = pl.BlockSpec((1, D_HID), lambda i: (0, 0))
    mat = pl.BlockSpec((D_HID, D_HID), lambda i: (0, 0))
    return pl.pallas_call(
        _layer2_body,
        grid=(_GRID,),
        in_specs=[pA, pB, blk, blk, vec, mat],
        out_specs=blk,
        out_shape=jax.ShapeDtypeStruct((N_NODES, D_HID), jnp.float32),
    )(raw, raw, g1, dinv, b1r, W2p)


def _final_body(ra_ref, rb_ref, g2_ref, dinv_ref, b2_ref, o_ref):
    o_ref[...] = (dinv_ref[...] * (ra_ref[0] + rb_ref[0] + g2_ref[...])
                  + b2_ref[...])


def _tc_final(raw, g2, dinv, b2p):
    blk = pl.BlockSpec((_BLK, D_HID), lambda i: (i, 0))
    pA = pl.BlockSpec((1, _BLK, D_HID), lambda i: (0, i, 0))
    pB = pl.BlockSpec((1, _BLK, D_HID), lambda i: (1, i, 0))
    vec = pl.BlockSpec((1, D_HID), lambda i: (0, 0))
    return pl.pallas_call(
        _final_body,
        grid=(_GRID,),
        in_specs=[pA, pB, blk, blk, vec],
        out_specs=blk,
        out_shape=jax.ShapeDtypeStruct((N_NODES, D_HID), jnp.float32),
    )(raw, raw, g2, dinv, b2p)


def kernel(x, edge_index, W1, b1, W2, b2):
    e = edge_index.shape[1]
    pad = E_PAD - e
    src_p = jnp.concatenate(
        [edge_index[0], jnp.zeros((pad,), edge_index.dtype)]).reshape(
            NW, KCH, BCH)
    dst_p = jnp.concatenate(
        [edge_index[1], jnp.full((pad,), N_NODES, edge_index.dtype)]).reshape(
            NW, KCH, BCH)
    W2p = jnp.zeros((D_HID, D_HID), jnp.float32).at[:, :D_OUT].set(W2)
    b1r = b1.reshape(1, D_HID)
    b2p = jnp.zeros((1, D_HID), jnp.float32).at[0, :D_OUT].set(b2)

    sc_degree = _make_sc_degree()
    sc_scatter = _make_sc_scatter()

    deg = sc_degree(dst_p)
    h1 = _tc_matmul1(x, W1)
    dinv, g1 = _tc_norm(deg, h1)
    raw1 = sc_scatter(g1, src_p, dst_p)
    g2 = _tc_layer2(raw1, g1, dinv, b1r, W2p)
    raw2 = sc_scatter(g2, src_p, dst_p)
    out16 = _tc_final(raw2, g2, dinv, b2p)
    return out16[:, :D_OUT]


# speedup vs baseline: 2.7959x; 1.7972x over previous
import functools

import jax
import jax.numpy as jnp
from jax import lax
from jax.experimental import pallas as pl
from jax.experimental.pallas import tpu as pltpu
from jax.experimental.pallas import tpu_sc as plsc

N_NODES = 10000
D_IN = 128
D_HID = 16
D_OUT = 10

NC = 2
NS = 16
NW = NC * NS
BCH = 1000
KCH = 10
GB = 2
NGRP = KCH // GB
NPAD = 10112
RPT = NPAD // NS
PK = 8
NP = N_NODES // PK
NPP = NPAD // PK
RPP = RPT // PK
GLAST = NP - (NS - 1) * RPP
GLAST16 = N_NODES - (NS - 1) * RPT


def _zero_my_slice(zbuf, table16, s):
    def zrow(i, _):
        zbuf[i] = jnp.zeros((D_HID,), jnp.float32)
        return 0

    lax.fori_loop(0, RPT, zrow, 0)
    pltpu.sync_copy(zbuf, table16.at[pl.ds(s * RPT, RPT)])


@functools.cache
def _make_sc_degree():
    mesh = plsc.VectorSubcoreMesh(core_axis_name="c", subcore_axis_name="s",
                                  num_cores=NC, num_subcores=NS)

    @functools.partial(
        pl.kernel,
        mesh=mesh,
        out_type=jax.ShapeDtypeStruct((NC, NPAD, D_HID), jnp.float32),
        scratch_types=[
            pltpu.VMEM((KCH, BCH), jnp.int32),
            pltpu.VMEM((BCH, D_HID), jnp.float32),
            pltpu.VMEM((RPT, D_HID), jnp.float32),
            pltpu.VMEM_SHARED((NPAD, D_HID), jnp.float32),
        ],
        compiler_params=pltpu.CompilerParams(use_tc_tiling_on_sc=False),
    )
    def sc_degree(eidx, out, didx, rows, zbuf, table16):
        c = lax.axis_index("c")
        s = lax.axis_index("s")
        wid = s * NC + c
        _zero_my_slice(zbuf, table16, s)

        def orow(i, _):
            rows[i] = jnp.ones((D_HID,), jnp.float32)
            return 0

        lax.fori_loop(0, BCH, orow, 0)
        pltpu.sync_copy(eidx.at[1, wid], didx)
        plsc.subcore_barrier()

        def chunk(j, _):
            pltpu.sync_copy(rows, table16.at[didx.at[j]], add=True)
            return 0

        lax.fori_loop(0, KCH, chunk, 0)
        plsc.subcore_barrier()
        pltpu.sync_copy(table16.at[pl.ds(s * RPT, RPT)],
                        out.at[c, pl.ds(s * RPT, RPT)])

    return sc_degree


@functools.cache
def _make_sc_scatter():
    mesh = plsc.VectorSubcoreMesh(core_axis_name="c", subcore_axis_name="s",
                                  num_cores=NC, num_subcores=NS)

    @functools.partial(
        pl.kernel,
        mesh=mesh,
        out_type=jax.ShapeDtypeStruct((NC, NPAD, D_HID), jnp.float32),
        scratch_types=[
            pltpu.VMEM((KCH, BCH), jnp.int32),
            pltpu.VMEM((KCH, BCH), jnp.int32),
            pltpu.VMEM((2, GB, BCH, D_HID), jnp.float32),
            pltpu.VMEM((RPT, D_HID), jnp.float32),
            pltpu.VMEM_SHARED((N_NODES, D_HID), jnp.float32),
            pltpu.VMEM_SHARED((NPAD, D_HID), jnp.float32),
            pltpu.SemaphoreType.DMA,
        ],
        compiler_params=pltpu.CompilerParams(use_tc_tiling_on_sc=False),
    )
    def sc_scatter(g_hbm, eidx, out, sidx, didx, rows, zbuf, gs16,
                   table16, gsem):
        c = lax.axis_index("c")
        s = lax.axis_index("s")
        wid = s * NC + c
        _zero_my_slice(zbuf, table16, s)

        @pl.when(s < NS - 1)
        def _():
            pltpu.sync_copy(g_hbm.at[pl.ds(s * RPT, RPT)],
                            gs16.at[pl.ds(s * RPT, RPT)])

        @pl.when(s == NS - 1)
        def _():
            pltpu.sync_copy(g_hbm.at[pl.ds((NS - 1) * RPT, GLAST16)],
                            gs16.at[pl.ds((NS - 1) * RPT, GLAST16)])

        pltpu.sync_copy(eidx.at[0, wid], sidx)
        pltpu.sync_copy(eidx.at[1, wid], didx)
        plsc.subcore_barrier()

        for b in range(GB):
            pltpu.async_copy(gs16.at[sidx.at[b]], rows.at[0, b], gsem)

        def group(g, _):
            h = lax.rem(g, 2)
            for b in range(GB):
                pltpu.make_async_copy(
                    g_hbm.at[pl.ds(0, BCH)], rows.at[h, b], gsem).wait()

            @pl.when(g + 1 < NGRP)
            def _():
                for b in range(GB):
                    pltpu.async_copy(
                        gs16.at[sidx.at[(g + 1) * GB + b]],
                        rows.at[1 - h, b], gsem)

            for b in range(GB):
                pltpu.sync_copy(rows.at[h, b],
                                table16.at[didx.at[g * GB + b]], add=True)
            return 0

        lax.fori_loop(0, NGRP, group, 0)
        plsc.subcore_barrier()
        pltpu.sync_copy(table16.at[pl.ds(s * RPT, RPT)],
                        out.at[c, pl.ds(s * RPT, RPT)])

    return sc_scatter


def _mm_norm_body(xp_ref, w1b_ref, deg_ref, dinv_ref, g_ref):
    hp = jnp.dot(xp_ref[...], w1b_ref[...],
                 preferred_element_type=jnp.float32)
    dinv = lax.rsqrt(deg_ref[0, :NP] + deg_ref[1, :NP] + 1.0)
    dinv_ref[...] = dinv
    g_ref[...] = hp * dinv


def _tc_mm_norm(xp, W1b, deg):
    return pl.pallas_call(
        _mm_norm_body,
        out_shape=[
            jax.ShapeDtypeStruct((NP, 128), jnp.float32),
            jax.ShapeDtypeStruct((NP, 128), jnp.float32),
        ],
    )(xp, W1b, deg)


def _layer2_body(raw_ref, g1_ref, dinv_ref, b1_ref, w2b_ref, g2_ref):
    pre = (dinv_ref[...] * (raw_ref[0, :NP] + raw_ref[1, :NP] + g1_ref[...])
           + b1_ref[...])
    h2 = jnp.maximum(pre, 0.0)
    g2 = jnp.dot(h2, w2b_ref[...], preferred_element_type=jnp.float32)
    g2_ref[...] = g2 * dinv_ref[...]


def _tc_layer2(raw, g1, dinv, b1p, W2b):
    return pl.pallas_call(
        _layer2_body,
        out_shape=jax.ShapeDtypeStruct((NP, 128), jnp.float32),
    )(raw, g1, dinv, b1p, W2b)


def _final_body(raw_ref, g2_ref, dinv_ref, b2_ref, o_ref):
    o_ref[...] = (dinv_ref[...] * (raw_ref[0, :NP] + raw_ref[1, :NP]
                                   + g2_ref[...]) + b2_ref[...])


def _tc_final(raw, g2, dinv, b2p):
    return pl.pallas_call(
        _final_body,
        out_shape=jax.ShapeDtypeStruct((NP, 128), jnp.float32),
    )(raw, g2, dinv, b2p)


def kernel(x, edge_index, W1, b1, W2, b2):
    eidx = edge_index.reshape(2, NW, KCH, BCH)
    eye8 = jnp.eye(PK, dtype=jnp.float32)
    W2p = jnp.zeros((D_HID, D_HID), jnp.float32).at[:, :D_OUT].set(W2)
    W1b = jnp.kron(eye8, W1)
    W2b = jnp.kron(eye8, W2p)
    b1p = jnp.tile(b1, PK).reshape(1, 128)
    b2p = jnp.tile(jnp.zeros((D_HID,), jnp.float32).at[:D_OUT].set(b2),
                   PK).reshape(1, 128)
    xp = x.reshape(NP, PK * D_IN)

    sc_degree = _make_sc_degree()
    sc_scatter = _make_sc_scatter()

    deg = sc_degree(eidx).reshape(NC, NPP, 128)
    dinv, g1p = _tc_mm_norm(xp, W1b, deg)
    g1 = g1p.reshape(N_NODES, D_HID)
    raw1 = sc_scatter(g1, eidx).reshape(NC, NPP, 128)
    g2p = _tc_layer2(raw1, g1p, dinv, b1p, W2b)
    raw2 = sc_scatter(g2p.reshape(N_NODES, D_HID), eidx).reshape(NC, NPP, 128)
    outp = _tc_final(raw2, g2p, dinv, b2p)
    return outp.reshape(N_NODES, D_HID)[:, :D_OUT]


# speedup vs baseline: 3.1455x; 1.1251x over previous
import functools

import jax
import jax.numpy as jnp
from jax import lax
from jax.experimental import pallas as pl
from jax.experimental.pallas import tpu as pltpu
from jax.experimental.pallas import tpu_sc as plsc

N_NODES = 10000
D_IN = 128
D_HID = 16
D_OUT = 10

NC = 2
NS = 16
NW = NC * NS
BCH = 1000
KCH = 10
GB = 2
NGRP = KCH // GB
NPAD = 10112
RPT = NPAD // NS
PK = 8
NP = N_NODES // PK
NPP = NPAD // PK
RPP = RPT // PK
GLAST = NP - (NS - 1) * RPP
GLAST16 = N_NODES - (NS - 1) * RPT


def _zero_my_slice(zbuf, table16, s):
    def zrow(i, _):
        zbuf[i] = jnp.zeros((D_HID,), jnp.float32)
        return 0

    lax.fori_loop(0, RPT, zrow, 0, unroll=8)
    pltpu.sync_copy(zbuf, table16.at[pl.ds(s * RPT, RPT)])


@functools.cache
def _make_sc_degree():
    mesh = plsc.VectorSubcoreMesh(core_axis_name="c", subcore_axis_name="s",
                                  num_cores=NC, num_subcores=NS)

    @functools.partial(
        pl.kernel,
        mesh=mesh,
        out_type=jax.ShapeDtypeStruct((NC, NPAD, D_HID), jnp.float32),
        scratch_types=[
            pltpu.VMEM((KCH, BCH), jnp.int32),
            pltpu.VMEM((BCH, D_HID), jnp.float32),
            pltpu.VMEM((RPT, D_HID), jnp.float32),
            pltpu.VMEM_SHARED((NPAD, D_HID), jnp.float32),
        ],
        compiler_params=pltpu.CompilerParams(use_tc_tiling_on_sc=False),
    )
    def sc_degree(eidx, out, didx, rows, zbuf, table16):
        c = lax.axis_index("c")
        s = lax.axis_index("s")
        wid = s * NC + c
        _zero_my_slice(zbuf, table16, s)

        def orow(i, _):
            rows[i] = jnp.ones((D_HID,), jnp.float32)
            return 0

        lax.fori_loop(0, BCH, orow, 0, unroll=8)
        pltpu.sync_copy(eidx.at[1, wid], didx)
        plsc.subcore_barrier()

        def chunk(j, _):
            pltpu.sync_copy(rows, table16.at[didx.at[j]], add=True)
            return 0

        lax.fori_loop(0, KCH, chunk, 0)
        plsc.subcore_barrier()
        pltpu.sync_copy(table16.at[pl.ds(s * RPT, RPT)],
                        out.at[c, pl.ds(s * RPT, RPT)])

    return sc_degree


@functools.cache
def _make_sc_scatter():
    mesh = plsc.VectorSubcoreMesh(core_axis_name="c", subcore_axis_name="s",
                                  num_cores=NC, num_subcores=NS)

    @functools.partial(
        pl.kernel,
        mesh=mesh,
        out_type=jax.ShapeDtypeStruct((NC, NPAD, D_HID), jnp.float32),
        scratch_types=[
            pltpu.VMEM((KCH, BCH), jnp.int32),
            pltpu.VMEM((KCH, BCH), jnp.int32),
            pltpu.VMEM((2, GB, BCH, D_HID), jnp.float32),
            pltpu.VMEM((RPT, D_HID), jnp.float32),
            pltpu.VMEM_SHARED((N_NODES, D_HID), jnp.float32),
            pltpu.VMEM_SHARED((NPAD, D_HID), jnp.float32),
            pltpu.SemaphoreType.DMA,
        ],
        compiler_params=pltpu.CompilerParams(use_tc_tiling_on_sc=False),
    )
    def sc_scatter(g_hbm, eidx, out, sidx, didx, rows, zbuf, gs16,
                   table16, gsem):
        c = lax.axis_index("c")
        s = lax.axis_index("s")
        wid = s * NC + c
        _zero_my_slice(zbuf, table16, s)

        @pl.when(s < NS - 1)
        def _():
            pltpu.sync_copy(g_hbm.at[pl.ds(s * RPT, RPT)],
                            gs16.at[pl.ds(s * RPT, RPT)])

        @pl.when(s == NS - 1)
        def _():
            pltpu.sync_copy(g_hbm.at[pl.ds((NS - 1) * RPT, GLAST16)],
                            gs16.at[pl.ds((NS - 1) * RPT, GLAST16)])

        pltpu.sync_copy(eidx.at[0, wid], sidx)
        pltpu.sync_copy(eidx.at[1, wid], didx)
        plsc.subcore_barrier()

        for b in range(GB):
            pltpu.async_copy(gs16.at[sidx.at[b]], rows.at[0, b], gsem)

        def group(g, _):
            h = lax.rem(g, 2)
            for b in range(GB):
                pltpu.make_async_copy(
                    g_hbm.at[pl.ds(0, BCH)], rows.at[h, b], gsem).wait()

            @pl.when(g + 1 < NGRP)
            def _():
                for b in range(GB):
                    pltpu.async_copy(
                        gs16.at[sidx.at[(g + 1) * GB + b]],
                        rows.at[1 - h, b], gsem)

            for b in range(GB):
                pltpu.sync_copy(rows.at[h, b],
                                table16.at[didx.at[g * GB + b]], add=True)
            return 0

        lax.fori_loop(0, NGRP, group, 0)
        plsc.subcore_barrier()
        pltpu.sync_copy(table16.at[pl.ds(s * RPT, RPT)],
                        out.at[c, pl.ds(s * RPT, RPT)])

    return sc_scatter


def _mm_norm_body(xp_ref, w1b_ref, deg_ref, dinv_ref, g_ref):
    hp = jnp.dot(xp_ref[...], w1b_ref[...],
                 preferred_element_type=jnp.float32)
    dinv = lax.rsqrt(deg_ref[0, :NP] + deg_ref[1, :NP] + 1.0)
    dinv_ref[...] = dinv
    g_ref[...] = hp * dinv


def _tc_mm_norm(xp, W1b, deg):
    return pl.pallas_call(
        _mm_norm_body,
        out_shape=[
            jax.ShapeDtypeStruct((NP, 128), jnp.float32),
            jax.ShapeDtypeStruct((NP, 128), jnp.float32),
        ],
    )(xp, W1b, deg)


def _layer2_body(raw_ref, g1_ref, dinv_ref, b1_ref, w2b_ref, g2_ref):
    pre = (dinv_ref[...] * (raw_ref[0, :NP] + raw_ref[1, :NP] + g1_ref[...])
           + b1_ref[...])
    h2 = jnp.maximum(pre, 0.0)
    g2 = jnp.dot(h2, w2b_ref[...], preferred_element_type=jnp.float32)
    g2_ref[...] = g2 * dinv_ref[...]


def _tc_layer2(raw, g1, dinv, b1p, W2b):
    return pl.pallas_call(
        _layer2_body,
        out_shape=jax.ShapeDtypeStruct((NP, 128), jnp.float32),
    )(raw, g1, dinv, b1p, W2b)


def _final_body(raw_ref, g2_ref, dinv_ref, b2_ref, o_ref):
    o_ref[...] = (dinv_ref[...] * (raw_ref[0, :NP] + raw_ref[1, :NP]
                                   + g2_ref[...]) + b2_ref[...])


def _tc_final(raw, g2, dinv, b2p):
    return pl.pallas_call(
        _final_body,
        out_shape=jax.ShapeDtypeStruct((NP, 128), jnp.float32),
    )(raw, g2, dinv, b2p)


def kernel(x, edge_index, W1, b1, W2, b2):
    eidx = edge_index.reshape(2, NW, KCH, BCH)
    eye8 = jnp.eye(PK, dtype=jnp.float32)
    W2p = jnp.zeros((D_HID, D_HID), jnp.float32).at[:, :D_OUT].set(W2)
    W1b = jnp.kron(eye8, W1)
    W2b = jnp.kron(eye8, W2p)
    b1p = jnp.tile(b1, PK).reshape(1, 128)
    b2p = jnp.tile(jnp.zeros((D_HID,), jnp.float32).at[:D_OUT].set(b2),
                   PK).reshape(1, 128)
    x_ob = lax.optimization_barrier((x, eidx))[0]
    xp = x_ob.reshape(NP, PK * D_IN)

    sc_degree = _make_sc_degree()
    sc_scatter = _make_sc_scatter()

    deg = sc_degree(eidx).reshape(NC, NPP, 128)
    dinv, g1p = _tc_mm_norm(xp, W1b, deg)
    g1 = g1p.reshape(N_NODES, D_HID)
    raw1 = sc_scatter(g1, eidx).reshape(NC, NPP, 128)
    g2p = _tc_layer2(raw1, g1p, dinv, b1p, W2b)
    raw2 = sc_scatter(g2p.reshape(N_NODES, D_HID), eidx).reshape(NC, NPP, 128)
    outp = _tc_final(raw2, g2p, dinv, b2p)
    return outp.reshape(N_NODES, D_HID)[:, :D_OUT]


# speedup vs baseline: 3.1926x; 1.0150x over previous
import functools

import jax
import jax.numpy as jnp
from jax import lax
from jax.experimental import pallas as pl
from jax.experimental.pallas import tpu as pltpu
from jax.experimental.pallas import tpu_sc as plsc

N_NODES = 10000
D_IN = 128
D_HID = 16
D_OUT = 10

NC = 2
NS = 16
NW = NC * NS
BCH = 1000
KCH = 10
GB = 2
NGRP = KCH // GB
NPAD = 10112
RPT = NPAD // NS
PK = 8
NP = N_NODES // PK
NPP = NPAD // PK
RPP = RPT // PK
GLAST = NP - (NS - 1) * RPP
GLAST16 = N_NODES - (NS - 1) * RPT


def _zero_my_slice(zbuf, table16, s):
    def zrow(i, _):
        zbuf[i] = jnp.zeros((D_HID,), jnp.float32)
        return 0

    lax.fori_loop(0, RPT, zrow, 0, unroll=8)
    pltpu.sync_copy(zbuf, table16.at[pl.ds(s * RPT, RPT)])


@functools.cache
def _make_sc_degree():
    mesh = plsc.VectorSubcoreMesh(core_axis_name="c", subcore_axis_name="s",
                                  num_cores=NC, num_subcores=NS)

    @functools.partial(
        pl.kernel,
        mesh=mesh,
        out_type=jax.ShapeDtypeStruct((NC, NPAD, D_HID), jnp.float32),
        scratch_types=[
            pltpu.VMEM((KCH, BCH), jnp.int32),
            pltpu.VMEM((BCH, D_HID), jnp.float32),
            pltpu.VMEM((RPT, D_HID), jnp.float32),
            pltpu.VMEM_SHARED((NPAD, D_HID), jnp.float32),
        ],
        compiler_params=pltpu.CompilerParams(use_tc_tiling_on_sc=False),
    )
    def sc_degree(eidx, out, didx, rows, zbuf, table16):
        c = lax.axis_index("c")
        s = lax.axis_index("s")
        wid = s * NC + c
        _zero_my_slice(zbuf, table16, s)

        def orow(i, _):
            rows[i] = jnp.ones((D_HID,), jnp.float32)
            return 0

        lax.fori_loop(0, BCH, orow, 0, unroll=8)
        pltpu.sync_copy(eidx.at[1, wid], didx)
        plsc.subcore_barrier()

        def chunk(j, _):
            pltpu.sync_copy(rows, table16.at[didx.at[j]], add=True)
            return 0

        lax.fori_loop(0, KCH, chunk, 0)
        plsc.subcore_barrier()
        pltpu.sync_copy(table16.at[pl.ds(s * RPT, RPT)],
                        out.at[c, pl.ds(s * RPT, RPT)])

    return sc_degree


@functools.cache
def _make_sc_scatter():
    mesh = plsc.VectorSubcoreMesh(core_axis_name="c", subcore_axis_name="s",
                                  num_cores=NC, num_subcores=NS)

    @functools.partial(
        pl.kernel,
        mesh=mesh,
        out_type=jax.ShapeDtypeStruct((NC, NPAD, D_HID), jnp.float32),
        scratch_types=[
            pltpu.VMEM((KCH, BCH), jnp.int32),
            pltpu.VMEM((KCH, BCH), jnp.int32),
            pltpu.VMEM((2, GB, BCH, D_HID), jnp.float32),
            pltpu.VMEM((RPT, D_HID), jnp.float32),
            pltpu.VMEM_SHARED((N_NODES, D_HID), jnp.float32),
            pltpu.VMEM_SHARED((NPAD, D_HID), jnp.float32),
            pltpu.SemaphoreType.DMA,
        ],
        compiler_params=pltpu.CompilerParams(use_tc_tiling_on_sc=False),
    )
    def sc_scatter(g_hbm, eidx, out, sidx, didx, rows, zbuf, gs16,
                   table16, gsem):
        c = lax.axis_index("c")
        s = lax.axis_index("s")
        wid = s * NC + c
        _zero_my_slice(zbuf, table16, s)

        @pl.when(s < NS - 1)
        def _():
            pltpu.sync_copy(g_hbm.at[pl.ds(s * RPT, RPT)],
                            gs16.at[pl.ds(s * RPT, RPT)])

        @pl.when(s == NS - 1)
        def _():
            pltpu.sync_copy(g_hbm.at[pl.ds((NS - 1) * RPT, GLAST16)],
                            gs16.at[pl.ds((NS - 1) * RPT, GLAST16)])

        pltpu.sync_copy(eidx.at[0, wid], sidx)
        pltpu.sync_copy(eidx.at[1, wid], didx)
        plsc.subcore_barrier()

        for b in range(GB):
            pltpu.async_copy(gs16.at[sidx.at[b]], rows.at[0, b], gsem)

        def group(g, _):
            h = lax.rem(g, 2)
            for b in range(GB):
                pltpu.make_async_copy(
                    g_hbm.at[pl.ds(0, BCH)], rows.at[h, b], gsem).wait()

            @pl.when(g + 1 < NGRP)
            def _():
                for b in range(GB):
                    pltpu.async_copy(
                        gs16.at[sidx.at[(g + 1) * GB + b]],
                        rows.at[1 - h, b], gsem)

            for b in range(GB):
                pltpu.sync_copy(rows.at[h, b],
                                table16.at[didx.at[g * GB + b]], add=True)
            return 0

        lax.fori_loop(0, NGRP, group, 0)
        plsc.subcore_barrier()
        pltpu.sync_copy(table16.at[pl.ds(s * RPT, RPT)],
                        out.at[c, pl.ds(s * RPT, RPT)])

    return sc_scatter


def _mm_body(xp_ref, w1b_ref, hp_ref):
    hp_ref[...] = jnp.dot(xp_ref[...], w1b_ref[...],
                          preferred_element_type=jnp.float32)


def _tc_mm(xp, W1b):
    return pl.pallas_call(
        _mm_body,
        out_shape=jax.ShapeDtypeStruct((NP, 128), jnp.float32),
    )(xp, W1b)


def _norm_body(hp_ref, deg_ref, dinv_ref, g_ref):
    dinv = lax.rsqrt(deg_ref[0, :NP] + deg_ref[1, :NP] + 1.0)
    dinv_ref[...] = dinv
    g_ref[...] = hp_ref[...] * dinv


def _tc_norm(hp, deg):
    return pl.pallas_call(
        _norm_body,
        out_shape=[
            jax.ShapeDtypeStruct((NP, 128), jnp.float32),
            jax.ShapeDtypeStruct((NP, 128), jnp.float32),
        ],
    )(hp, deg)


def _layer2_body(raw_ref, g1_ref, dinv_ref, b1_ref, w2b_ref, g2_ref):
    pre = (dinv_ref[...] * (raw_ref[0, :NP] + raw_ref[1, :NP] + g1_ref[...])
           + b1_ref[...])
    h2 = jnp.maximum(pre, 0.0)
    g2 = jnp.dot(h2, w2b_ref[...], preferred_element_type=jnp.float32)
    g2_ref[...] = g2 * dinv_ref[...]


def _tc_layer2(raw, g1, dinv, b1p, W2b):
    return pl.pallas_call(
        _layer2_body,
        out_shape=jax.ShapeDtypeStruct((NP, 128), jnp.float32),
    )(raw, g1, dinv, b1p, W2b)


def _final_body(raw_ref, g2_ref, dinv_ref, b2_ref, o_ref):
    o_ref[...] = (dinv_ref[...] * (raw_ref[0, :NP] + raw_ref[1, :NP]
                                   + g2_ref[...]) + b2_ref[...])


def _tc_final(raw, g2, dinv, b2p):
    return pl.pallas_call(
        _final_body,
        out_shape=jax.ShapeDtypeStruct((NP, 128), jnp.float32),
    )(raw, g2, dinv, b2p)


def kernel(x, edge_index, W1, b1, W2, b2):
    eidx = edge_index.reshape(2, NW, KCH, BCH)
    eye8 = jnp.eye(PK, dtype=jnp.float32)
    W2p = jnp.zeros((D_HID, D_HID), jnp.float32).at[:, :D_OUT].set(W2)
    W1b = jnp.kron(eye8, W1)
    W2b = jnp.kron(eye8, W2p)
    b1p = jnp.tile(b1, PK).reshape(1, 128)
    b2p = jnp.tile(jnp.zeros((D_HID,), jnp.float32).at[:D_OUT].set(b2),
                   PK).reshape(1, 128)
    x_ob = lax.optimization_barrier((x, eidx))[0]
    xp = x_ob.reshape(NP, PK * D_IN)

    sc_degree = _make_sc_degree()
    sc_scatter = _make_sc_scatter()

    deg = sc_degree(eidx).reshape(NC, NPP, 128)
    hp = _tc_mm(xp, W1b)
    dinv, g1p = _tc_norm(hp, deg)
    g1 = g1p.reshape(N_NODES, D_HID)
    raw1 = sc_scatter(g1, eidx).reshape(NC, NPP, 128)
    g2p = _tc_layer2(raw1, g1p, dinv, b1p, W2b)
    raw2 = sc_scatter(g2p.reshape(N_NODES, D_HID), eidx).reshape(NC, NPP, 128)
    outp = _tc_final(raw2, g2p, dinv, b2p)
    return outp.reshape(N_NODES, D_HID)[:, :D_OUT]
